# Initial kernel scaffold; baseline (speedup 1.0000x reference)
import jax
import jax.numpy as jnp
from jax.experimental import pallas as pl


def kernel(ngrams, ngrams_weights, embedding, W_h, b_h, W_s, b_s):
    raise NotImplementedError("write your pallas kernel here")


# speedup vs baseline: 3.5052x; 3.5052x over previous
import functools

import jax
import jax.numpy as jnp
from jax import lax
from jax.experimental import pallas as pl
from jax.experimental.pallas import tpu as pltpu
from jax.experimental.pallas import tpu_sc as plsc

EMB = 32
HALF = 16
SEG = 208
GCH = 104


def _sc_pooled_embedding(ng_flat, w_flat, embedding):
    B, PER_B = ng_flat.shape
    orders = PER_B // SEG
    n_gath = PER_B // GCH
    nw = 32
    items_per_w = B // nw

    mesh = plsc.VectorSubcoreMesh(core_axis_name="c", subcore_axis_name="s")

    @functools.partial(
        pl.kernel,
        mesh=mesh,
        compiler_params=pltpu.CompilerParams(use_tc_tiling_on_sc=False),
        out_type=jax.ShapeDtypeStruct((B, orders * EMB), jnp.float32),
        scratch_types=[
            pltpu.VMEM((PER_B,), jnp.int32),
            pltpu.VMEM((PER_B,), jnp.float32),
            pltpu.VMEM((PER_B, EMB), jnp.float32),
            pltpu.VMEM((orders * EMB,), jnp.float32),
            pltpu.SemaphoreType.DMA,
        ],
    )
    def k(ng_hbm, w_hbm, emb_hbm, out_hbm, idx_v, w_v, rows_v, out_v, sem):
        wid = lax.axis_index("s") * 2 + lax.axis_index("c")

        def item_body(i, carry):
            item = wid * items_per_w + i
            pltpu.sync_copy(ng_hbm.at[item], idx_v)
            pltpu.sync_copy(w_hbm.at[item], w_v)
            cps = []
            for g in range(n_gath):
                cps.append(
                    pltpu.async_copy(
                        emb_hbm.at[idx_v.at[pl.ds(g * GCH, GCH)]],
                        rows_v.at[pl.ds(g * GCH, GCH)],
                        sem,
                    )
                )
            for cp in cps:
                cp.wait()

            for o in range(orders):
                def grp_body(g, acc):
                    acc_lo, acc_hi = acc
                    base = o * SEG + g * HALF
                    w16 = w_v[pl.ds(base, HALF)]
                    for t in range(HALF):
                        wv = jnp.full((HALF,), w16[t], jnp.float32)
                        r_lo = rows_v[base + t, pl.ds(0, HALF)]
                        r_hi = rows_v[base + t, pl.ds(HALF, HALF)]
                        acc_lo = acc_lo + wv * r_lo
                        acc_hi = acc_hi + wv * r_hi
                    return (acc_lo, acc_hi)

                z = jnp.zeros((HALF,), jnp.float32)
                acc_lo, acc_hi = lax.fori_loop(0, SEG // HALF, grp_body, (z, z))
                out_v[pl.ds(o * EMB, HALF)] = acc_lo * 0.25
                out_v[pl.ds(o * EMB + HALF, HALF)] = acc_hi * 0.25

            pltpu.sync_copy(out_v, out_hbm.at[item])
            return carry

        lax.fori_loop(0, items_per_w, item_body, 0)

    return k(ng_flat, w_flat, embedding)


def _mlp_logsoftmax(embed, W_h, b_h, W_s, b_s):
    B, D = embed.shape
    HID = W_h.shape[1]
    LAB = W_s.shape[1]
    BM = 512

    def body(x_ref, wh_ref, bh_ref, ws_ref, bs_ref, out_ref):
        x = x_ref[...]
        h = jnp.dot(x, wh_ref[...], preferred_element_type=jnp.float32) + bh_ref[...]
        logits = jnp.dot(h, ws_ref[...], preferred_element_type=jnp.float32) + bs_ref[...]
        m = jnp.max(logits, axis=-1, keepdims=True)
        s = logits - m
        lse = jnp.log(jnp.sum(jnp.exp(s), axis=-1, keepdims=True))
        out_ref[...] = s - lse

    return pl.pallas_call(
        body,
        grid=(B // BM,),
        in_specs=[
            pl.BlockSpec((BM, D), lambda i: (i, 0)),
            pl.BlockSpec((D, HID), lambda i: (0, 0)),
            pl.BlockSpec((1, HID), lambda i: (0, 0)),
            pl.BlockSpec((HID, LAB), lambda i: (0, 0)),
            pl.BlockSpec((1, LAB), lambda i: (0, 0)),
        ],
        out_specs=pl.BlockSpec((BM, LAB), lambda i: (i, 0)),
        out_shape=jax.ShapeDtypeStruct((B, LAB), jnp.float32),
    )(embed, W_h, b_h.reshape(1, HID), W_s, b_s.reshape(1, LAB))


def kernel(ngrams, ngrams_weights, embedding, W_h, b_h, W_s, b_s):
    B, orders, ngr, hsh = ngrams.shape
    per_o = ngr * hsh
    pad = SEG - per_o
    ng = ngrams.reshape(B, orders, per_o).astype(jnp.int32)
    wt = ngrams_weights.reshape(B, orders, per_o)
    ng = jnp.pad(ng, ((0, 0), (0, 0), (0, pad)))
    wt = jnp.pad(wt, ((0, 0), (0, 0), (0, pad)))
    ng_flat = ng.reshape(B, orders * SEG)
    w_flat = wt.reshape(B, orders * SEG)
    embed = _sc_pooled_embedding(ng_flat, w_flat, embedding)
    return _mlp_logsoftmax(embed, W_h, b_h, W_s, b_s)
